# speedup vs baseline: 2.3713x; 2.3713x over previous
import functools

import jax
import jax.numpy as jnp
from jax.experimental import pallas as pl
from jax.experimental.pallas import tpu as pltpu

EPS = 1e-12


def _vq_kernel(z_ref, v_in_ref, g_in_ref, b_in_ref, cb_ref, v_out_ref,
               g_out_ref, b_out_ref, out_ref, idx_ref, *, batch, n_codes):
    f32 = jnp.float32
    v_in = v_in_ref[...]
    w_in = g_in_ref[...] * v_in / jnp.maximum(
        jnp.sqrt(jnp.sum(v_in * v_in, axis=1, keepdims=True)), EPS)
    v_out = v_out_ref[...]
    w_out = g_out_ref[...] * v_out / jnp.maximum(
        jnp.sqrt(jnp.sum(v_out * v_out, axis=1, keepdims=True)), EPS)
    cb = cb_ref[...]
    cb_n = cb / jnp.maximum(
        jnp.sqrt(jnp.sum(cb * cb, axis=1, keepdims=True)), EPS)
    cb_bias = -0.5 * jnp.sum(cb_n * cb_n, axis=1, keepdims=True)

    tt = z_ref.shape[-1]
    iota_k = jax.lax.broadcasted_iota(jnp.int32, (n_codes, tt), 0)

    for b in range(batch):
        zb = z_ref[b]
        ze = jnp.dot(w_in, zb, preferred_element_type=f32) + b_in_ref[...]
        nsq = jnp.sum(ze * ze, axis=0, keepdims=True)
        zen = ze / jnp.maximum(jnp.sqrt(nsq), EPS)
        scores = jnp.dot(cb_n, zen, preferred_element_type=f32) + cb_bias
        mx = jnp.max(scores, axis=0, keepdims=True)
        idx = jnp.min(jnp.where(scores == mx, iota_k, n_codes), axis=0)
        onehot = (iota_k == idx[None, :]).astype(f32)
        zq = jax.lax.dot_general(cb, onehot, (((0,), (0,)), ((), ())),
                                 preferred_element_type=f32)
        outb = jnp.dot(w_out, zq, preferred_element_type=f32) + b_out_ref[...]
        out_ref[b] = outb
        idx_ref[b, :] = idx


def kernel(z, v_in, g_in, b_in, codebook, v_out, g_out, b_out):
    B, Din, T = z.shape
    K, Dc = codebook.shape
    TT = 256
    grid = (T // TT,)

    full = lambda shape: pl.BlockSpec(shape, lambda t: (0,) * len(shape))
    out, idx = pl.pallas_call(
        functools.partial(_vq_kernel, batch=B, n_codes=K),
        grid=grid,
        in_specs=[
            pl.BlockSpec((B, Din, TT), lambda t: (0, 0, t)),
            full((Dc, Din)),
            full((Dc, 1)),
            full((Dc, 1)),
            full((K, Dc)),
            full((Din, Dc)),
            full((Din, 1)),
            full((Din, 1)),
        ],
        out_specs=[
            pl.BlockSpec((B, Din, TT), lambda t: (0, 0, t)),
            pl.BlockSpec((B, TT), lambda t: (0, t)),
        ],
        out_shape=[
            jax.ShapeDtypeStruct((B, Din, T), jnp.float32),
            jax.ShapeDtypeStruct((B, T), jnp.int32),
        ],
        compiler_params=pltpu.CompilerParams(
            dimension_semantics=("arbitrary",)),
    )(z, v_in, g_in.reshape(Dc, 1), b_in.reshape(Dc, 1), codebook,
      v_out, g_out.reshape(Din, 1), b_out.reshape(Din, 1))
    return out, idx
